# baseline (speedup 1.0000x reference)
import functools

import numpy as np
import jax
import jax.numpy as jnp
from jax import lax
from jax.experimental import pallas as pl
from jax.experimental.pallas import tpu as pltpu
from jax.experimental.pallas import tpu_sc as plsc

MIN_TEMPERATURE = np.float32(1e-8)
LOG_MIN_PROB = np.float32(np.log(np.float32(1e-10)))
INT_MIN = np.int32(-2**31)
INT_MAX = np.int32(2**31 - 1)
ROW_BLOCK = 8
B, V = 128, 100000
CHUNK = 8192
NCHUNK = 13
VP = CHUNK * NCHUNK
NBKT = 2048
BSHIFT = 21
CAP = 8192
NW = 32
ROWS_PER_W = B // NW


def _floor_avg(lo, hi):
    return (lo >> 1) + (hi >> 1) + (lo & hi & 1)


def _fold(ibits):
    return ibits ^ ((ibits >> 31) & np.int32(0x7FFFFFFF))


def _prep_body(logits_ref, t_ref, x_ref, m_ref):
    rb = logits_ref.shape[0]
    t = t_ref[0, 0, :].reshape(rb, 1)
    x = logits_ref[...] / t
    x = x + np.float32(0.0)
    x_ref[...] = x
    m_ref[0, 0, :] = jnp.max(x, axis=-1)


def _sc_iota():
    return lax.iota(jnp.int32, 16)


def _splat_i(v):
    return jnp.full((16,), 0, jnp.int32) + v


def _sload(ref, i):
    v = ref[pl.ds(i, 16)]
    z = jnp.zeros((16,), v.dtype)
    return jnp.sum(jnp.where(_sc_iota() == 0, v, z))


def _select_body(x_hbm, m_hbm, keff_hbm, topp_hbm, out_hbm,
                 xb, hcnt, hsum, mcnt, msum, sufcnt, sufsum,
                 keybuf, idxbuf, sm, skeff, stp, resbuf):
    wid = lax.axis_index("s") * 2 + lax.axis_index("c")
    pltpu.sync_copy(m_hbm, sm.at[pl.ds(0, B)])
    pltpu.sync_copy(keff_hbm, skeff.at[pl.ds(0, B)])
    pltpu.sync_copy(topp_hbm, stp.at[pl.ds(0, B)])
    lanes = _sc_iota()
    ones_i = jnp.full((16,), 1, jnp.int32)
    zeros_f = jnp.zeros((16,), jnp.float32)
    zeros_i = jnp.zeros((16,), jnp.int32)

    def row_pass(row, vreg_fn, carry_init):
        def chunk_body(c, carry):
            pltpu.sync_copy(x_hbm.at[row, pl.ds(c * CHUNK, CHUNK)], xb)

            def vbody(v, cc):
                xv = xb[pl.ds(v * 16, 16)]
                return vreg_fn(c * CHUNK + v * 16, xv, cc)

            return lax.fori_loop(0, CHUNK // 16, vbody, carry)

        return lax.fori_loop(0, NCHUNK, chunk_body, carry_init)

    def process_row(r, _):
        row = wid * ROWS_PER_W + r
        m_row = _sload(sm, row)
        keff = _sload(skeff, row)
        topp = _sload(stp, row)

        def clr(i, _c):
            hcnt[pl.ds(i * 16, 16)] = zeros_i
            hsum[pl.ds(i * 16, 16)] = zeros_f
            return 0

        lax.fori_loop(0, (NBKT * 16) // 16, clr, 0)

        def hist_fn(base, xv, c):
            e = jnp.exp(xv - m_row)
            sk = _fold(lax.bitcast_convert_type(xv, jnp.int32))
            ub = lax.shift_right_logical(sk ^ INT_MIN, BSHIFT)
            addr = lanes * NBKT + ub
            plsc.addupdate_scatter(hcnt, [addr], ones_i)
            plsc.addupdate_scatter(hsum, [addr], e)
            return c

        row_pass(row, hist_fn, 0)

        def merge(ci, _c):
            ac = zeros_i
            asm = zeros_f
            for rep in range(16):
                ac = ac + hcnt[pl.ds(rep * NBKT + ci * 16, 16)]
                asm = asm + hsum[pl.ds(rep * NBKT + ci * 16, 16)]
            mcnt[pl.ds(ci * 16, 16)] = ac
            msum[pl.ds(ci * 16, 16)] = asm
            return 0

        lax.fori_loop(0, NBKT // 16, merge, 0)

        def sfx(cj, carry):
            rc, rs = carry
            ci = NBKT // 16 - 1 - cj
            cv = lax.rev(mcnt[pl.ds(ci * 16, 16)], (0,))
            sv = lax.rev(msum[pl.ds(ci * 16, 16)], (0,))
            cc = plsc.cumsum(cv)
            cs = plsc.cumsum(sv)
            sufcnt[pl.ds(ci * 16, 16)] = lax.rev(rc + cc - cv, (0,))
            sufsum[pl.ds(ci * 16, 16)] = lax.rev(rs + cs - sv, (0,))
            return rc + jnp.max(cc), rs + jnp.max(cs)

        lax.fori_loop(0, NBKT // 16, sfx, (np.int32(0), np.float32(0.0)))

        def bk_scan(ci, bk):
            sc_v = sufcnt[pl.ds(ci * 16, 16)]
            bidx = ci * 16 + lanes
            cand = jnp.where(sc_v < keff, bidx, NBKT)
            return jnp.minimum(bk, jnp.min(cand))

        bk = lax.fori_loop(0, NBKT // 16, bk_scan, np.int32(NBKT))
        n_gt_above = _sload(sufcnt, bk)
        sum_above = _sload(sufsum, bk)

        def compact(bkt):
            def cfn(base, xv, coff):
                sk = _fold(lax.bitcast_convert_type(xv, jnp.int32))
                ub = lax.shift_right_logical(sk ^ INT_MIN, BSHIFT)
                gidx = base + lanes
                mask = (ub == bkt) & (gidx < V) & (coff < CAP)
                plsc.store_compressed(keybuf.at[pl.ds(coff, 16)], sk,
                                      mask=mask)
                plsc.store_compressed(idxbuf.at[pl.ds(coff, 16)], gidx,
                                      mask=mask)
                return coff + jnp.sum(mask.astype(jnp.int32))

            return row_pass(row, cfn, np.int32(0))

        ck = compact(bk)
        ktrips = (ck + 15) >> 4

        def masked_count_gt(mid):
            def body(i, acc):
                kv = keybuf[pl.ds(i * 16, 16)]
                valid = (i * 16 + lanes) < ck
                return acc + jnp.where((kv > mid) & valid, ones_i, zeros_i)

            return jnp.sum(lax.fori_loop(0, ktrips, body, zeros_i))

        lo0 = (bk << BSHIFT) ^ INT_MIN
        hi0 = lo0 | np.int32((1 << BSHIFT) - 1)

        def tk_it(_, lh):
            lo, hi = lh
            mid = _floor_avg(lo, hi)
            pred = (n_gt_above + masked_count_gt(mid)) >= keff
            return jnp.where(pred, mid + 1, lo), jnp.where(pred, hi, mid)

        tk, _ = lax.fori_loop(0, BSHIFT, tk_it, (lo0, hi0))
        r_k = keff - (n_gt_above + masked_count_gt(tk))

        def sgt_in(i, acc):
            kv = keybuf[pl.ds(i * 16, 16)]
            valid = (i * 16 + lanes) < ck
            ib = _fold(kv)
            ev = jnp.exp(lax.bitcast_convert_type(ib, jnp.float32) - m_row)
            return acc + jnp.where((kv > tk) & valid, ev, zeros_f)

        sum_gt_tk = jnp.sum(lax.fori_loop(0, ktrips, sgt_in, zeros_f))

        def jscan(tkey, rwant, kept_extra_tk, jk_arg):
            def body(i, carry):
                bs, jfound = carry
                kv = keybuf[pl.ds(i * 16, 16)]
                iv = idxbuf[pl.ds(i * 16, 16)]
                valid = (i * 16 + lanes) < ck
                keep1 = (kv > kept_extra_tk) | ((kv == kept_extra_tk) &
                                                (iv <= jk_arg))
                eq = (kv == tkey) & valid & keep1
                pc = plsc.cumsum(eq.astype(jnp.int32)) + bs
                hit = eq & (pc == rwant)
                jf = jnp.max(jnp.where(hit, iv, -1))
                return jnp.max(pc), jnp.maximum(jfound, jf)

            _, j = lax.fori_loop(0, ktrips, body, (np.int32(0), np.int32(-1)))
            return j

        jk = jscan(tk, r_k, tk, INT_MAX)

        e_tk = jnp.max(jnp.exp(
            lax.bitcast_convert_type(_fold(_splat_i(tk)), jnp.float32)
            - m_row))
        z1 = sum_above + sum_gt_tk + r_k.astype(jnp.float32) * e_tk
        thresh = topp * z1

        def bp_scan(ci, bp):
            sv = sufsum[pl.ds(ci * 16, 16)]
            mc = mcnt[pl.ds(ci * 16, 16)]
            bidx = ci * 16 + lanes
            pred = (sv <= thresh) & ((mc > 0) | (bidx == bk)) & (bidx >= bk)
            cand = jnp.where(pred, bidx, NBKT)
            return jnp.minimum(bp, jnp.min(cand))

        bp = lax.fori_loop(0, NBKT // 16, bp_scan, np.int32(NBKT))
        sum_above_p = _sload(sufsum, bp)

        cp = compact(bp)
        ptrips = (cp + 15) >> 4

        def masked_sum_gt_kept(mid):
            def body(i, acc):
                kv = keybuf[pl.ds(i * 16, 16)]
                iv = idxbuf[pl.ds(i * 16, 16)]
                valid = (i * 16 + lanes) < cp
                keep1 = (kv > tk) | ((kv == tk) & (iv <= jk))
                ib = _fold(kv)
                ev = jnp.exp(lax.bitcast_convert_type(ib, jnp.float32)
                             - m_row)
                return acc + jnp.where((kv > mid) & valid & keep1, ev,
                                       zeros_f)

            return jnp.sum(lax.fori_loop(0, ptrips, body, zeros_f))

        lo0p = (bp << BSHIFT) ^ INT_MIN
        hi0p = lo0p | np.int32((1 << BSHIFT) - 1)

        def tp_it(_, lh):
            lo, hi = lh
            mid = _floor_avg(lo, hi)
            pred = (sum_above_p + masked_sum_gt_kept(mid)) <= thresh
            return jnp.where(pred, lo, mid + 1), jnp.where(pred, mid, hi)

        tp, _ = lax.fori_loop(0, BSHIFT, tp_it, (lo0p, hi0p))

        d_t = sum_above_p + masked_sum_gt_kept(tp)

        def neq_kept(i, acc):
            kv = keybuf[pl.ds(i * 16, 16)]
            iv = idxbuf[pl.ds(i * 16, 16)]
            valid = (i * 16 + lanes) < cp
            keep1 = (kv > tk) | ((kv == tk) & (iv <= jk))
            return acc + jnp.where((kv == tp) & valid & keep1, ones_i,
                                   zeros_i)

        n_eq_p = jnp.sum(lax.fori_loop(0, ptrips, neq_kept, zeros_i))
        e_tp = jnp.max(jnp.exp(
            lax.bitcast_convert_type(_fold(_splat_i(tp)), jnp.float32)
            - m_row))
        q = jnp.max((zeros_f + (thresh - d_t)) / (zeros_f + e_tp))
        r_p = jnp.minimum(q, n_eq_p.astype(jnp.float32)).astype(jnp.int32) + 1
        r_p = jnp.minimum(r_p, n_eq_p)
        r_p = jnp.where(e_tp > np.float32(0.0), r_p, n_eq_p)

        def jscan_p():
            def body(i, carry):
                bs, jfound = carry
                kv = keybuf[pl.ds(i * 16, 16)]
                iv = idxbuf[pl.ds(i * 16, 16)]
                valid = (i * 16 + lanes) < cp
                keep1 = (kv > tk) | ((kv == tk) & (iv <= jk))
                eq = (kv == tp) & valid & keep1
                pc = plsc.cumsum(eq.astype(jnp.int32)) + bs
                hit = eq & (pc == r_p)
                jf = jnp.max(jnp.where(hit, iv, -1))
                return jnp.max(pc), jnp.maximum(jfound, jf)

            _, j = lax.fori_loop(0, ptrips, body,
                                 (np.int32(0), np.int32(-1)))
            return j

        jp = jscan_p()

        vals = jnp.where(lanes % 4 == 0, tk,
                         jnp.where(lanes % 4 == 1, jk,
                                   jnp.where(lanes % 4 == 2, tp, jp)))
        plsc.store_scatter(resbuf, [jnp.minimum(r * 4 + lanes, 15)], vals,
                           mask=lanes < 4)
        return 0

    lax.fori_loop(0, ROWS_PER_W, process_row, 0)
    pltpu.sync_copy(resbuf, out_hbm.at[wid])


def _final_body(x_ref, g_ref, m_ref, tk_ref, jk_ref, tp_ref, jp_ref, mp_ref,
                out_ref):
    rb, vp = x_ref.shape
    m = m_ref[0, 0, :].reshape(rb, 1)
    tk = tk_ref[0, 0, :].reshape(rb, 1)
    jk = jk_ref[0, 0, :].reshape(rb, 1)
    tp = tp_ref[0, 0, :].reshape(rb, 1)
    jp = jp_ref[0, 0, :].reshape(rb, 1)
    min_p = mp_ref[0, 0, :].reshape(rb, 1)

    x = x_ref[...]
    skey = _fold(lax.bitcast_convert_type(x, jnp.int32))
    iota = lax.broadcasted_iota(jnp.int32, (rb, vp), 1)
    e = jnp.exp(x - m)

    kept2 = ((skey > tk) | ((skey == tk) & (iota <= jk))) & \
            ((skey > tp) | ((skey == tp) & (iota <= jp)))
    z2 = jnp.sum(jnp.where(kept2, e, np.float32(0.0)), axis=-1,
                 keepdims=True)
    thr = min_p * (np.float32(1.0) / z2)
    kept3 = kept2 & jnp.logical_not((e / z2) < thr)
    z3 = jnp.sum(jnp.where(kept3, e, np.float32(0.0)), axis=-1,
                 keepdims=True)
    lz3 = jnp.log(z3)
    lp = jnp.where(kept3, jnp.maximum(x - m - lz3, LOG_MIN_PROB),
                   LOG_MIN_PROB)
    f = g_ref[...] + lp
    fmax = jnp.max(f, axis=-1, keepdims=True)
    tok = jnp.min(jnp.where(f == fmax, iota, vp), axis=-1)
    out_ref[0, 0, :] = tok


def _run(logits, t, top_ps, top_ks, min_ps, g):
    rb = ROW_BLOCK
    nblk = B // rb

    def r3(a, dtype):
        return a.astype(dtype).reshape(nblk, 1, rb)

    row_spec = pl.BlockSpec((rb, VP), lambda i: (i, 0))
    s_spec = pl.BlockSpec((1, 1, rb), lambda i: (i, 0, 0))

    logits_p = jnp.pad(logits, ((0, 0), (0, VP - V)),
                       constant_values=-np.inf)

    x_pad, m3 = pl.pallas_call(
        _prep_body,
        grid=(nblk,),
        in_specs=[row_spec, s_spec],
        out_specs=[row_spec, s_spec],
        out_shape=[jax.ShapeDtypeStruct((B, VP), jnp.float32),
                   jax.ShapeDtypeStruct((nblk, 1, rb), jnp.float32)],
    )(logits_p, r3(t, jnp.float32))
    m = m3.reshape(B)

    k_eff = jnp.where((top_ks > 0) & (top_ks < V), top_ks, V)

    mesh = plsc.VectorSubcoreMesh(core_axis_name="c", subcore_axis_name="s")
    sel = pl.kernel(
        _select_body,
        mesh=mesh,
        out_type=jax.ShapeDtypeStruct((NW, 16), jnp.int32),
        compiler_params=pltpu.CompilerParams(needs_layout_passes=False),
        scratch_types=[
            pltpu.VMEM((CHUNK,), jnp.float32),
            pltpu.VMEM((NBKT * 16,), jnp.int32),
            pltpu.VMEM((NBKT * 16,), jnp.float32),
            pltpu.VMEM((NBKT,), jnp.int32),
            pltpu.VMEM((NBKT,), jnp.float32),
            pltpu.VMEM((NBKT + 16,), jnp.int32),
            pltpu.VMEM((NBKT + 16,), jnp.float32),
            pltpu.VMEM((CAP + 16,), jnp.int32),
            pltpu.VMEM((CAP + 16,), jnp.int32),
            pltpu.VMEM((B + 16,), jnp.float32),
            pltpu.VMEM((B + 16,), jnp.int32),
            pltpu.VMEM((B + 16,), jnp.float32),
            pltpu.VMEM((16,), jnp.int32),
        ],
    )(x_pad, m, k_eff.astype(jnp.int32), top_ps.astype(jnp.float32))

    sel = sel.reshape(B, 4)
    tk, jk, tp, jp = sel[:, 0], sel[:, 1], sel[:, 2], sel[:, 3]

    g_pad = jnp.pad(g, ((0, 0), (0, VP - V)), constant_values=0.0)
    out = pl.pallas_call(
        _final_body,
        grid=(nblk,),
        in_specs=[row_spec, row_spec, s_spec, s_spec, s_spec, s_spec,
                  s_spec, s_spec],
        out_specs=s_spec,
        out_shape=jax.ShapeDtypeStruct((nblk, 1, rb), jnp.int32),
    )(x_pad, g_pad, m3, r3(tk, jnp.int32), r3(jk, jnp.int32),
      r3(tp, jnp.int32), r3(jp, jnp.int32), r3(min_ps, jnp.float32))
    return out.reshape(B)


def kernel(logits, temperatures, top_ps, top_ks, min_ps):
    t = jnp.maximum(temperatures, MIN_TEMPERATURE)
    g = jax.random.gumbel(jax.random.key(123), (B, V), jnp.float32)
    return _run(logits.astype(jnp.float32), t, top_ps, top_ks, min_ps, g)


# speedup vs baseline: 1.0745x; 1.0745x over previous
import numpy as np
import jax
import jax.numpy as jnp
from jax import lax
from jax.experimental import pallas as pl
from jax.experimental.pallas import tpu as pltpu
from jax.experimental.pallas import tpu_sc as plsc

MIN_TEMPERATURE = np.float32(1e-8)
LOG_MIN_PROB = np.float32(np.log(np.float32(1e-10)))
INT_MIN = np.int32(-2**31)
INT_MAX = np.int32(2**31 - 1)
ROW_BLOCK = 8
B, V = 128, 100000
CHUNK = 8192
NCHUNK = 13
VP = CHUNK * NCHUNK
NBKT = 2048
BSHIFT = 21
CAP = 8192
NW = 32
ROWS_PER_W = B // NW


def _floor_avg(lo, hi):
    return (lo >> 1) + (hi >> 1) + (lo & hi & 1)


def _fold(ibits):
    return ibits ^ ((ibits >> 31) & np.int32(0x7FFFFFFF))


def _prep_body(logits_ref, t_ref, x_ref, m_ref):
    rb = logits_ref.shape[0]
    t = t_ref[0, 0, :].reshape(rb, 1)
    x = logits_ref[...] / t
    x = x + np.float32(0.0)
    x_ref[...] = x
    m_ref[0, 0, :] = jnp.max(x, axis=-1)


def _sc_iota():
    return lax.iota(jnp.int32, 16)


def _splat_i(v):
    return jnp.full((16,), 0, jnp.int32) + v


def _sload(ref, i):
    v = ref[pl.ds(i, 16)]
    z = jnp.zeros((16,), v.dtype)
    return jnp.sum(jnp.where(_sc_iota() == 0, v, z))


def _select_body(x_hbm, m_hbm, keff_hbm, topp_hbm, out_hbm,
                 xb, xb2, hcnt, hsum, mcnt, msum, sufcnt, sufsum,
                 keybuf, idxbuf, sm, skeff, stp, resbuf, dsem, dsem2):
    wid = lax.axis_index("s") * 2 + lax.axis_index("c")
    pltpu.sync_copy(m_hbm, sm.at[pl.ds(0, B)])
    pltpu.sync_copy(keff_hbm, skeff.at[pl.ds(0, B)])
    pltpu.sync_copy(topp_hbm, stp.at[pl.ds(0, B)])
    lanes = _sc_iota()
    ones_i = jnp.full((16,), 1, jnp.int32)
    zeros_f = jnp.zeros((16,), jnp.float32)
    zeros_i = jnp.zeros((16,), jnp.int32)

    def row_pass(row, vreg_fn, carry_init):
        bufs = (xb, xb2)
        sems = (dsem, dsem2)

        def start(c):
            return pltpu.async_copy(
                x_hbm.at[row, pl.ds(c * CHUNK, CHUNK)], bufs[c % 2],
                sems[c % 2])

        copy = start(0)
        carry = carry_init
        for c in range(NCHUNK):
            nxt = start(c + 1) if c + 1 < NCHUNK else None
            copy.wait()
            buf = bufs[c % 2]

            def vbody(v, cc, c=c, buf=buf):
                xv = buf[pl.ds(v * 16, 16)]
                return vreg_fn(c * CHUNK + v * 16, xv, cc)

            carry = lax.fori_loop(0, CHUNK // 16, vbody, carry)
            copy = nxt
        return carry

    def process_row(r, _):
        row = wid * ROWS_PER_W + r
        m_row = _sload(sm, row)
        keff = _sload(skeff, row)
        topp = _sload(stp, row)

        def clr(i, _c):
            hcnt[pl.ds(i * 16, 16)] = zeros_i
            hsum[pl.ds(i * 16, 16)] = zeros_f
            return 0

        lax.fori_loop(0, (NBKT * 16) // 16, clr, 0)

        def hist_fn(base, xv, c):
            e = jnp.exp(xv - m_row)
            sk = _fold(lax.bitcast_convert_type(xv, jnp.int32))
            ub = lax.shift_right_logical(sk ^ INT_MIN, BSHIFT)
            addr = lanes * NBKT + ub
            plsc.addupdate_scatter(hcnt, [addr], ones_i)
            plsc.addupdate_scatter(hsum, [addr], e)
            return c

        row_pass(row, hist_fn, 0)

        def merge(ci, _c):
            ac = zeros_i
            asm = zeros_f
            for rep in range(16):
                ac = ac + hcnt[pl.ds(rep * NBKT + ci * 16, 16)]
                asm = asm + hsum[pl.ds(rep * NBKT + ci * 16, 16)]
            mcnt[pl.ds(ci * 16, 16)] = ac
            msum[pl.ds(ci * 16, 16)] = asm
            return 0

        lax.fori_loop(0, NBKT // 16, merge, 0)

        def sfx(cj, carry):
            rc, rs = carry
            ci = NBKT // 16 - 1 - cj
            cv = lax.rev(mcnt[pl.ds(ci * 16, 16)], (0,))
            sv = lax.rev(msum[pl.ds(ci * 16, 16)], (0,))
            cc = plsc.cumsum(cv)
            cs = plsc.cumsum(sv)
            sufcnt[pl.ds(ci * 16, 16)] = lax.rev(rc + cc - cv, (0,))
            sufsum[pl.ds(ci * 16, 16)] = lax.rev(rs + cs - sv, (0,))
            return rc + jnp.max(cc), rs + jnp.max(cs)

        lax.fori_loop(0, NBKT // 16, sfx, (np.int32(0), np.float32(0.0)))

        def bk_scan(ci, bk):
            sc_v = sufcnt[pl.ds(ci * 16, 16)]
            bidx = ci * 16 + lanes
            cand = jnp.where(sc_v < keff, bidx, NBKT)
            return jnp.minimum(bk, jnp.min(cand))

        bk = lax.fori_loop(0, NBKT // 16, bk_scan, np.int32(NBKT))
        n_gt_above = _sload(sufcnt, bk)
        sum_above = _sload(sufsum, bk)

        def compact(bkt):
            def cfn(base, xv, coff):
                sk = _fold(lax.bitcast_convert_type(xv, jnp.int32))
                ub = lax.shift_right_logical(sk ^ INT_MIN, BSHIFT)
                gidx = base + lanes
                mask = (ub == bkt) & (gidx < V) & (coff < CAP)
                plsc.store_compressed(keybuf.at[pl.ds(coff, 16)], sk,
                                      mask=mask)
                plsc.store_compressed(idxbuf.at[pl.ds(coff, 16)], gidx,
                                      mask=mask)
                return coff + jnp.sum(mask.astype(jnp.int32))

            return row_pass(row, cfn, np.int32(0))

        ck = compact(bk)
        ktrips = (ck + 15) >> 4

        def masked_count_gt(mid):
            def body(i, acc):
                kv = keybuf[pl.ds(i * 16, 16)]
                valid = (i * 16 + lanes) < ck
                return acc + jnp.where((kv > mid) & valid, ones_i, zeros_i)

            return jnp.sum(lax.fori_loop(0, ktrips, body, zeros_i))

        lo0 = (bk << BSHIFT) ^ INT_MIN
        hi0 = lo0 | np.int32((1 << BSHIFT) - 1)

        def tk_it(_, lh):
            lo, hi = lh
            mid = _floor_avg(lo, hi)
            pred = (n_gt_above + masked_count_gt(mid)) >= keff
            return jnp.where(pred, mid + 1, lo), jnp.where(pred, hi, mid)

        tk, _ = lax.fori_loop(0, BSHIFT, tk_it, (lo0, hi0))
        r_k = keff - (n_gt_above + masked_count_gt(tk))

        def sgt_in(i, acc):
            kv = keybuf[pl.ds(i * 16, 16)]
            valid = (i * 16 + lanes) < ck
            ib = _fold(kv)
            ev = jnp.exp(lax.bitcast_convert_type(ib, jnp.float32) - m_row)
            return acc + jnp.where((kv > tk) & valid, ev, zeros_f)

        sum_gt_tk = jnp.sum(lax.fori_loop(0, ktrips, sgt_in, zeros_f))

        def jscan(tkey, rwant, kept_extra_tk, jk_arg):
            def body(i, carry):
                bs, jfound = carry
                kv = keybuf[pl.ds(i * 16, 16)]
                iv = idxbuf[pl.ds(i * 16, 16)]
                valid = (i * 16 + lanes) < ck
                keep1 = (kv > kept_extra_tk) | ((kv == kept_extra_tk) &
                                                (iv <= jk_arg))
                eq = (kv == tkey) & valid & keep1
                pc = plsc.cumsum(eq.astype(jnp.int32)) + bs
                hit = eq & (pc == rwant)
                jf = jnp.max(jnp.where(hit, iv, -1))
                return jnp.max(pc), jnp.maximum(jfound, jf)

            _, j = lax.fori_loop(0, ktrips, body, (np.int32(0), np.int32(-1)))
            return j

        jk = jscan(tk, r_k, tk, INT_MAX)

        e_tk = jnp.max(jnp.exp(
            lax.bitcast_convert_type(_fold(_splat_i(tk)), jnp.float32)
            - m_row))
        z1 = sum_above + sum_gt_tk + r_k.astype(jnp.float32) * e_tk
        thresh = topp * z1

        def bp_scan(ci, bp):
            sv = sufsum[pl.ds(ci * 16, 16)]
            mc = mcnt[pl.ds(ci * 16, 16)]
            bidx = ci * 16 + lanes
            pred = (sv <= thresh) & ((mc > 0) | (bidx == bk)) & (bidx >= bk)
            cand = jnp.where(pred, bidx, NBKT)
            return jnp.minimum(bp, jnp.min(cand))

        bp = lax.fori_loop(0, NBKT // 16, bp_scan, np.int32(NBKT))
        sum_above_p = _sload(sufsum, bp)

        cp = compact(bp)
        ptrips = (cp + 15) >> 4

        def masked_sum_gt_kept(mid):
            def body(i, acc):
                kv = keybuf[pl.ds(i * 16, 16)]
                iv = idxbuf[pl.ds(i * 16, 16)]
                valid = (i * 16 + lanes) < cp
                keep1 = (kv > tk) | ((kv == tk) & (iv <= jk))
                ib = _fold(kv)
                ev = jnp.exp(lax.bitcast_convert_type(ib, jnp.float32)
                             - m_row)
                return acc + jnp.where((kv > mid) & valid & keep1, ev,
                                       zeros_f)

            return jnp.sum(lax.fori_loop(0, ptrips, body, zeros_f))

        lo0p = (bp << BSHIFT) ^ INT_MIN
        hi0p = lo0p | np.int32((1 << BSHIFT) - 1)

        def tp_it(_, lh):
            lo, hi = lh
            mid = _floor_avg(lo, hi)
            pred = (sum_above_p + masked_sum_gt_kept(mid)) <= thresh
            return jnp.where(pred, lo, mid + 1), jnp.where(pred, mid, hi)

        tp, _ = lax.fori_loop(0, BSHIFT, tp_it, (lo0p, hi0p))

        d_t = sum_above_p + masked_sum_gt_kept(tp)

        def neq_kept(i, acc):
            kv = keybuf[pl.ds(i * 16, 16)]
            iv = idxbuf[pl.ds(i * 16, 16)]
            valid = (i * 16 + lanes) < cp
            keep1 = (kv > tk) | ((kv == tk) & (iv <= jk))
            return acc + jnp.where((kv == tp) & valid & keep1, ones_i,
                                   zeros_i)

        n_eq_p = jnp.sum(lax.fori_loop(0, ptrips, neq_kept, zeros_i))
        e_tp = jnp.max(jnp.exp(
            lax.bitcast_convert_type(_fold(_splat_i(tp)), jnp.float32)
            - m_row))
        q = jnp.max((zeros_f + (thresh - d_t)) / (zeros_f + e_tp))
        r_p = jnp.minimum(q, n_eq_p.astype(jnp.float32)).astype(jnp.int32) + 1
        r_p = jnp.minimum(r_p, n_eq_p)
        r_p = jnp.where(e_tp > np.float32(0.0), r_p, n_eq_p)

        def jscan_p():
            def body(i, carry):
                bs, jfound = carry
                kv = keybuf[pl.ds(i * 16, 16)]
                iv = idxbuf[pl.ds(i * 16, 16)]
                valid = (i * 16 + lanes) < cp
                keep1 = (kv > tk) | ((kv == tk) & (iv <= jk))
                eq = (kv == tp) & valid & keep1
                pc = plsc.cumsum(eq.astype(jnp.int32)) + bs
                hit = eq & (pc == r_p)
                jf = jnp.max(jnp.where(hit, iv, -1))
                return jnp.max(pc), jnp.maximum(jfound, jf)

            _, j = lax.fori_loop(0, ptrips, body,
                                 (np.int32(0), np.int32(-1)))
            return j

        jp = jscan_p()

        vals = jnp.where(lanes % 4 == 0, tk,
                         jnp.where(lanes % 4 == 1, jk,
                                   jnp.where(lanes % 4 == 2, tp, jp)))
        plsc.store_scatter(resbuf, [jnp.minimum(r * 4 + lanes, 15)], vals,
                           mask=lanes < 4)
        return 0

    lax.fori_loop(0, ROWS_PER_W, process_row, 0)
    pltpu.sync_copy(resbuf, out_hbm.at[wid])


def _final_body(x_ref, g_ref, m_ref, tk_ref, jk_ref, tp_ref, jp_ref, mp_ref,
                out_ref):
    rb, vp = x_ref.shape
    m = m_ref[0, 0, :].reshape(rb, 1)
    tk = tk_ref[0, 0, :].reshape(rb, 1)
    jk = jk_ref[0, 0, :].reshape(rb, 1)
    tp = tp_ref[0, 0, :].reshape(rb, 1)
    jp = jp_ref[0, 0, :].reshape(rb, 1)
    min_p = mp_ref[0, 0, :].reshape(rb, 1)

    x = x_ref[...]
    skey = _fold(lax.bitcast_convert_type(x, jnp.int32))
    iota = lax.broadcasted_iota(jnp.int32, (rb, vp), 1)
    e = jnp.exp(x - m)

    kept2 = ((skey > tk) | ((skey == tk) & (iota <= jk))) & \
            ((skey > tp) | ((skey == tp) & (iota <= jp)))
    z2 = jnp.sum(jnp.where(kept2, e, np.float32(0.0)), axis=-1,
                 keepdims=True)
    thr = min_p * (np.float32(1.0) / z2)
    kept3 = kept2 & jnp.logical_not((e / z2) < thr)
    z3 = jnp.sum(jnp.where(kept3, e, np.float32(0.0)), axis=-1,
                 keepdims=True)
    lz3 = jnp.log(z3)
    lp = jnp.where(kept3, jnp.maximum(x - m - lz3, LOG_MIN_PROB),
                   LOG_MIN_PROB)
    f = g_ref[...] + lp
    fmax = jnp.max(f, axis=-1, keepdims=True)
    tok = jnp.min(jnp.where(f == fmax, iota, vp), axis=-1)
    out_ref[0, 0, :] = tok


def _run(logits, t, top_ps, top_ks, min_ps, g):
    rb = ROW_BLOCK
    nblk = B // rb

    def r3(a, dtype):
        return a.astype(dtype).reshape(nblk, 1, rb)

    row_spec = pl.BlockSpec((rb, VP), lambda i: (i, 0))
    s_spec = pl.BlockSpec((1, 1, rb), lambda i: (i, 0, 0))

    logits_p = jnp.pad(logits, ((0, 0), (0, VP - V)),
                       constant_values=-np.inf)

    x_pad, m3 = pl.pallas_call(
        _prep_body,
        grid=(nblk,),
        in_specs=[row_spec, s_spec],
        out_specs=[row_spec, s_spec],
        out_shape=[jax.ShapeDtypeStruct((B, VP), jnp.float32),
                   jax.ShapeDtypeStruct((nblk, 1, rb), jnp.float32)],
    )(logits_p, r3(t, jnp.float32))
    m = m3.reshape(B)

    k_eff = jnp.where((top_ks > 0) & (top_ks < V), top_ks, V)

    mesh = plsc.VectorSubcoreMesh(core_axis_name="c", subcore_axis_name="s")
    sel = pl.kernel(
        _select_body,
        mesh=mesh,
        out_type=jax.ShapeDtypeStruct((NW, 16), jnp.int32),
        compiler_params=pltpu.CompilerParams(needs_layout_passes=False),
        scratch_types=[
            pltpu.VMEM((CHUNK,), jnp.float32),
            pltpu.VMEM((CHUNK,), jnp.float32),
            pltpu.VMEM((NBKT * 16,), jnp.int32),
            pltpu.VMEM((NBKT * 16,), jnp.float32),
            pltpu.VMEM((NBKT,), jnp.int32),
            pltpu.VMEM((NBKT,), jnp.float32),
            pltpu.VMEM((NBKT + 16,), jnp.int32),
            pltpu.VMEM((NBKT + 16,), jnp.float32),
            pltpu.VMEM((CAP + 16,), jnp.int32),
            pltpu.VMEM((CAP + 16,), jnp.int32),
            pltpu.VMEM((B + 16,), jnp.float32),
            pltpu.VMEM((B + 16,), jnp.int32),
            pltpu.VMEM((B + 16,), jnp.float32),
            pltpu.VMEM((16,), jnp.int32),
            pltpu.SemaphoreType.DMA,
            pltpu.SemaphoreType.DMA,
        ],
    )(x_pad, m, k_eff.astype(jnp.int32), top_ps.astype(jnp.float32))

    sel = sel.reshape(B, 4)
    tk, jk, tp, jp = sel[:, 0], sel[:, 1], sel[:, 2], sel[:, 3]

    g_pad = jnp.pad(g, ((0, 0), (0, VP - V)), constant_values=0.0)
    out = pl.pallas_call(
        _final_body,
        grid=(nblk,),
        in_specs=[row_spec, row_spec, s_spec, s_spec, s_spec, s_spec,
                  s_spec, s_spec],
        out_specs=s_spec,
        out_shape=jax.ShapeDtypeStruct((nblk, 1, rb), jnp.int32),
    )(x_pad, g_pad, m3, r3(tk, jnp.int32), r3(jk, jnp.int32),
      r3(tp, jnp.int32), r3(jp, jnp.int32), r3(min_ps, jnp.float32))
    return out.reshape(B)


def kernel(logits, temperatures, top_ps, top_ks, min_ps):
    t = jnp.maximum(temperatures, MIN_TEMPERATURE)
    g = jax.random.gumbel(jax.random.key(123), (B, V), jnp.float32)
    return _run(logits.astype(jnp.float32), t, top_ps, top_ks, min_ps, g)


# speedup vs baseline: 1.1002x; 1.0239x over previous
import numpy as np
import jax
import jax.numpy as jnp
from jax import lax
from jax.experimental import pallas as pl
from jax.experimental.pallas import tpu as pltpu
from jax.experimental.pallas import tpu_sc as plsc

MIN_TEMPERATURE = np.float32(1e-8)
LOG_MIN_PROB = np.float32(np.log(np.float32(1e-10)))
INT_MIN = np.int32(-2**31)
INT_MAX = np.int32(2**31 - 1)
ROW_BLOCK = 8
B, V = 128, 100000
CHUNK = 8192
NCHUNK = 13
VP = CHUNK * NCHUNK
NBKT = 2048
BSHIFT = 21
CAP = 8192
NW = 32
ROWS_PER_W = B // NW


def _floor_avg(lo, hi):
    return (lo >> 1) + (hi >> 1) + (lo & hi & 1)


def _fold(ibits):
    return ibits ^ ((ibits >> 31) & np.int32(0x7FFFFFFF))


def _prep_body(logits_ref, t_ref, x_ref, m_ref):
    rb = logits_ref.shape[0]
    t = t_ref[0, 0, :].reshape(rb, 1)
    x = logits_ref[...] / t
    x = x + np.float32(0.0)
    x_ref[...] = x
    m_ref[0, 0, :] = jnp.max(x, axis=-1)


def _sc_iota():
    return lax.iota(jnp.int32, 16)


def _splat_i(v):
    return jnp.full((16,), 0, jnp.int32) + v


def _sload(ref, i):
    v = ref[pl.ds(i, 16)]
    z = jnp.zeros((16,), v.dtype)
    return jnp.sum(jnp.where(_sc_iota() == 0, v, z))


def _select_body(x_hbm, m_hbm, keff_hbm, topp_hbm, out_hbm,
                 xb, xb2, hcnt, hsum, mcnt, msum, sufcnt, sufsum,
                 keybuf, idxbuf, sm, skeff, stp, resbuf, dsem, dsem2):
    wid = lax.axis_index("s") * 2 + lax.axis_index("c")
    pltpu.sync_copy(m_hbm, sm.at[pl.ds(0, B)])
    pltpu.sync_copy(keff_hbm, skeff.at[pl.ds(0, B)])
    pltpu.sync_copy(topp_hbm, stp.at[pl.ds(0, B)])
    lanes = _sc_iota()
    ones_i = jnp.full((16,), 1, jnp.int32)
    zeros_f = jnp.zeros((16,), jnp.float32)
    zeros_i = jnp.zeros((16,), jnp.int32)

    def row_pass(row, vreg_fn, carry_init):
        bufs = (xb, xb2)
        sems = (dsem, dsem2)

        def start(c):
            return pltpu.async_copy(
                x_hbm.at[row, pl.ds(c * CHUNK, CHUNK)], bufs[c % 2],
                sems[c % 2])

        copy = start(0)
        carry = carry_init
        for c in range(NCHUNK):
            nxt = start(c + 1) if c + 1 < NCHUNK else None
            copy.wait()
            buf = bufs[c % 2]

            def vbody(v, cc, c=c, buf=buf):
                xv = buf[pl.ds(v * 16, 16)]
                return vreg_fn(c * CHUNK + v * 16, xv, cc)

            carry = lax.fori_loop(0, CHUNK // 16, vbody, carry)
            copy = nxt
        return carry

    def process_row(r, _):
        row = wid * ROWS_PER_W + r
        m_row = _sload(sm, row)
        keff = _sload(skeff, row)
        topp = _sload(stp, row)

        def clr(i, _c):
            hcnt[pl.ds(i * 16, 16)] = zeros_i
            hsum[pl.ds(i * 16, 16)] = zeros_f
            return 0

        lax.fori_loop(0, (NBKT * 16) // 16, clr, 0)

        def hist_fn(base, xv, c):
            e = jnp.exp(xv - m_row)
            sk = _fold(lax.bitcast_convert_type(xv, jnp.int32))
            ub = lax.shift_right_logical(sk ^ INT_MIN, BSHIFT)
            addr = ub * 16 + lanes
            plsc.addupdate_scatter(hcnt, [addr], ones_i)
            plsc.addupdate_scatter(hsum, [addr], e)
            return c

        row_pass(row, hist_fn, 0)

        def merge(ci, _c):
            ac = zeros_i
            asm = zeros_f
            for j in range(16):
                b = ci * 16 + j
                cs = jnp.sum(hcnt[pl.ds(b * 16, 16)])
                ss = jnp.sum(hsum[pl.ds(b * 16, 16)])
                ac = jnp.where(lanes == j, cs, ac)
                asm = jnp.where(lanes == j, ss, asm)
            mcnt[pl.ds(ci * 16, 16)] = ac
            msum[pl.ds(ci * 16, 16)] = asm
            return 0

        lax.fori_loop(0, NBKT // 16, merge, 0)

        def sfx(cj, carry):
            rc, rs = carry
            ci = NBKT // 16 - 1 - cj
            cv = lax.rev(mcnt[pl.ds(ci * 16, 16)], (0,))
            sv = lax.rev(msum[pl.ds(ci * 16, 16)], (0,))
            cc = plsc.cumsum(cv)
            cs = plsc.cumsum(sv)
            sufcnt[pl.ds(ci * 16, 16)] = lax.rev(rc + cc - cv, (0,))
            sufsum[pl.ds(ci * 16, 16)] = lax.rev(rs + cs - sv, (0,))
            return rc + jnp.max(cc), rs + jnp.max(cs)

        lax.fori_loop(0, NBKT // 16, sfx, (np.int32(0), np.float32(0.0)))

        def bk_scan(ci, bk):
            sc_v = sufcnt[pl.ds(ci * 16, 16)]
            bidx = ci * 16 + lanes
            cand = jnp.where(sc_v < keff, bidx, NBKT)
            return jnp.minimum(bk, jnp.min(cand))

        bk = lax.fori_loop(0, NBKT // 16, bk_scan, np.int32(NBKT))
        n_gt_above = _sload(sufcnt, bk)
        sum_above = _sload(sufsum, bk)

        def compact(bkt):
            def cfn(base, xv, coff):
                sk = _fold(lax.bitcast_convert_type(xv, jnp.int32))
                ub = lax.shift_right_logical(sk ^ INT_MIN, BSHIFT)
                gidx = base + lanes
                mask = (ub == bkt) & (gidx < V) & (coff < CAP)
                plsc.store_compressed(keybuf.at[pl.ds(coff, 16)], sk,
                                      mask=mask)
                plsc.store_compressed(idxbuf.at[pl.ds(coff, 16)], gidx,
                                      mask=mask)
                return coff + jnp.sum(mask.astype(jnp.int32))

            return row_pass(row, cfn, np.int32(0))

        ck = compact(bk)
        ktrips = (ck + 15) >> 4

        def masked_count_gt(mid):
            def body(i, acc):
                kv = keybuf[pl.ds(i * 16, 16)]
                valid = (i * 16 + lanes) < ck
                return acc + jnp.where((kv > mid) & valid, ones_i, zeros_i)

            return jnp.sum(lax.fori_loop(0, ktrips, body, zeros_i))

        lo0 = (bk << BSHIFT) ^ INT_MIN
        hi0 = lo0 | np.int32((1 << BSHIFT) - 1)

        def tk_it(_, lh):
            lo, hi = lh
            mid = _floor_avg(lo, hi)
            pred = (n_gt_above + masked_count_gt(mid)) >= keff
            return jnp.where(pred, mid + 1, lo), jnp.where(pred, hi, mid)

        tk, _ = lax.fori_loop(0, BSHIFT, tk_it, (lo0, hi0))
        r_k = keff - (n_gt_above + masked_count_gt(tk))

        def sgt_in(i, acc):
            kv = keybuf[pl.ds(i * 16, 16)]
            valid = (i * 16 + lanes) < ck
            ib = _fold(kv)
            ev = jnp.exp(lax.bitcast_convert_type(ib, jnp.float32) - m_row)
            return acc + jnp.where((kv > tk) & valid, ev, zeros_f)

        sum_gt_tk = jnp.sum(lax.fori_loop(0, ktrips, sgt_in, zeros_f))

        def jscan(tkey, rwant, kept_extra_tk, jk_arg):
            def body(i, carry):
                bs, jfound = carry
                kv = keybuf[pl.ds(i * 16, 16)]
                iv = idxbuf[pl.ds(i * 16, 16)]
                valid = (i * 16 + lanes) < ck
                keep1 = (kv > kept_extra_tk) | ((kv == kept_extra_tk) &
                                                (iv <= jk_arg))
                eq = (kv == tkey) & valid & keep1
                pc = plsc.cumsum(eq.astype(jnp.int32)) + bs
                hit = eq & (pc == rwant)
                jf = jnp.max(jnp.where(hit, iv, -1))
                return jnp.max(pc), jnp.maximum(jfound, jf)

            _, j = lax.fori_loop(0, ktrips, body, (np.int32(0), np.int32(-1)))
            return j

        jk = jscan(tk, r_k, tk, INT_MAX)

        e_tk = jnp.max(jnp.exp(
            lax.bitcast_convert_type(_fold(_splat_i(tk)), jnp.float32)
            - m_row))
        z1 = sum_above + sum_gt_tk + r_k.astype(jnp.float32) * e_tk
        thresh = topp * z1

        def bp_scan(ci, bp):
            sv = sufsum[pl.ds(ci * 16, 16)]
            mc = mcnt[pl.ds(ci * 16, 16)]
            bidx = ci * 16 + lanes
            pred = (sv <= thresh) & ((mc > 0) | (bidx == bk)) & (bidx >= bk)
            cand = jnp.where(pred, bidx, NBKT)
            return jnp.minimum(bp, jnp.min(cand))

        bp = lax.fori_loop(0, NBKT // 16, bp_scan, np.int32(NBKT))
        sum_above_p = _sload(sufsum, bp)

        cp = compact(bp)
        ptrips = (cp + 15) >> 4

        def masked_sum_gt_kept(mid):
            def body(i, acc):
                kv = keybuf[pl.ds(i * 16, 16)]
                iv = idxbuf[pl.ds(i * 16, 16)]
                valid = (i * 16 + lanes) < cp
                keep1 = (kv > tk) | ((kv == tk) & (iv <= jk))
                ib = _fold(kv)
                ev = jnp.exp(lax.bitcast_convert_type(ib, jnp.float32)
                             - m_row)
                return acc + jnp.where((kv > mid) & valid & keep1, ev,
                                       zeros_f)

            return jnp.sum(lax.fori_loop(0, ptrips, body, zeros_f))

        lo0p = (bp << BSHIFT) ^ INT_MIN
        hi0p = lo0p | np.int32((1 << BSHIFT) - 1)

        def tp_it(_, lh):
            lo, hi = lh
            mid = _floor_avg(lo, hi)
            pred = (sum_above_p + masked_sum_gt_kept(mid)) <= thresh
            return jnp.where(pred, lo, mid + 1), jnp.where(pred, mid, hi)

        tp, _ = lax.fori_loop(0, BSHIFT, tp_it, (lo0p, hi0p))

        d_t = sum_above_p + masked_sum_gt_kept(tp)

        def neq_kept(i, acc):
            kv = keybuf[pl.ds(i * 16, 16)]
            iv = idxbuf[pl.ds(i * 16, 16)]
            valid = (i * 16 + lanes) < cp
            keep1 = (kv > tk) | ((kv == tk) & (iv <= jk))
            return acc + jnp.where((kv == tp) & valid & keep1, ones_i,
                                   zeros_i)

        n_eq_p = jnp.sum(lax.fori_loop(0, ptrips, neq_kept, zeros_i))
        e_tp = jnp.max(jnp.exp(
            lax.bitcast_convert_type(_fold(_splat_i(tp)), jnp.float32)
            - m_row))
        q = jnp.max((zeros_f + (thresh - d_t)) / (zeros_f + e_tp))
        r_p = jnp.minimum(q, n_eq_p.astype(jnp.float32)).astype(jnp.int32) + 1
        r_p = jnp.minimum(r_p, n_eq_p)
        r_p = jnp.where(e_tp > np.float32(0.0), r_p, n_eq_p)

        def jscan_p():
            def body(i, carry):
                bs, jfound = carry
                kv = keybuf[pl.ds(i * 16, 16)]
                iv = idxbuf[pl.ds(i * 16, 16)]
                valid = (i * 16 + lanes) < cp
                keep1 = (kv > tk) | ((kv == tk) & (iv <= jk))
                eq = (kv == tp) & valid & keep1
                pc = plsc.cumsum(eq.astype(jnp.int32)) + bs
                hit = eq & (pc == r_p)
                jf = jnp.max(jnp.where(hit, iv, -1))
                return jnp.max(pc), jnp.maximum(jfound, jf)

            _, j = lax.fori_loop(0, ptrips, body,
                                 (np.int32(0), np.int32(-1)))
            return j

        jp = jscan_p()

        vals = jnp.where(lanes % 4 == 0, tk,
                         jnp.where(lanes % 4 == 1, jk,
                                   jnp.where(lanes % 4 == 2, tp, jp)))
        plsc.store_scatter(resbuf, [jnp.minimum(r * 4 + lanes, 15)], vals,
                           mask=lanes < 4)
        return 0

    lax.fori_loop(0, ROWS_PER_W, process_row, 0)
    pltpu.sync_copy(resbuf, out_hbm.at[wid])


def _final_body(x_ref, g_ref, m_ref, tk_ref, jk_ref, tp_ref, jp_ref, mp_ref,
                out_ref):
    rb, vp = x_ref.shape
    m = m_ref[0, 0, :].reshape(rb, 1)
    tk = tk_ref[0, 0, :].reshape(rb, 1)
    jk = jk_ref[0, 0, :].reshape(rb, 1)
    tp = tp_ref[0, 0, :].reshape(rb, 1)
    jp = jp_ref[0, 0, :].reshape(rb, 1)
    min_p = mp_ref[0, 0, :].reshape(rb, 1)

    x = x_ref[...]
    skey = _fold(lax.bitcast_convert_type(x, jnp.int32))
    iota = lax.broadcasted_iota(jnp.int32, (rb, vp), 1)
    e = jnp.exp(x - m)

    kept2 = ((skey > tk) | ((skey == tk) & (iota <= jk))) & \
            ((skey > tp) | ((skey == tp) & (iota <= jp)))
    z2 = jnp.sum(jnp.where(kept2, e, np.float32(0.0)), axis=-1,
                 keepdims=True)
    thr = min_p * (np.float32(1.0) / z2)
    kept3 = kept2 & jnp.logical_not((e / z2) < thr)
    z3 = jnp.sum(jnp.where(kept3, e, np.float32(0.0)), axis=-1,
                 keepdims=True)
    lz3 = jnp.log(z3)
    lp = jnp.where(kept3, jnp.maximum(x - m - lz3, LOG_MIN_PROB),
                   LOG_MIN_PROB)
    f = g_ref[...] + lp
    fmax = jnp.max(f, axis=-1, keepdims=True)
    tok = jnp.min(jnp.where(f == fmax, iota, vp), axis=-1)
    out_ref[0, 0, :] = tok


def _run(logits, t, top_ps, top_ks, min_ps, g):
    rb = ROW_BLOCK
    nblk = B // rb

    def r3(a, dtype):
        return a.astype(dtype).reshape(nblk, 1, rb)

    row_spec = pl.BlockSpec((rb, VP), lambda i: (i, 0))
    s_spec = pl.BlockSpec((1, 1, rb), lambda i: (i, 0, 0))

    logits_p = jnp.pad(logits, ((0, 0), (0, VP - V)),
                       constant_values=-np.inf)

    x_pad, m3 = pl.pallas_call(
        _prep_body,
        grid=(nblk,),
        in_specs=[row_spec, s_spec],
        out_specs=[row_spec, s_spec],
        out_shape=[jax.ShapeDtypeStruct((B, VP), jnp.float32),
                   jax.ShapeDtypeStruct((nblk, 1, rb), jnp.float32)],
    )(logits_p, r3(t, jnp.float32))
    m = m3.reshape(B)

    k_eff = jnp.where((top_ks > 0) & (top_ks < V), top_ks, V)

    mesh = plsc.VectorSubcoreMesh(core_axis_name="c", subcore_axis_name="s")
    sel = pl.kernel(
        _select_body,
        mesh=mesh,
        out_type=jax.ShapeDtypeStruct((NW, 16), jnp.int32),
        compiler_params=pltpu.CompilerParams(needs_layout_passes=False),
        scratch_types=[
            pltpu.VMEM((CHUNK,), jnp.float32),
            pltpu.VMEM((CHUNK,), jnp.float32),
            pltpu.VMEM((NBKT * 16,), jnp.int32),
            pltpu.VMEM((NBKT * 16,), jnp.float32),
            pltpu.VMEM((NBKT,), jnp.int32),
            pltpu.VMEM((NBKT,), jnp.float32),
            pltpu.VMEM((NBKT + 16,), jnp.int32),
            pltpu.VMEM((NBKT + 16,), jnp.float32),
            pltpu.VMEM((CAP + 16,), jnp.int32),
            pltpu.VMEM((CAP + 16,), jnp.int32),
            pltpu.VMEM((B + 16,), jnp.float32),
            pltpu.VMEM((B + 16,), jnp.int32),
            pltpu.VMEM((B + 16,), jnp.float32),
            pltpu.VMEM((16,), jnp.int32),
            pltpu.SemaphoreType.DMA,
            pltpu.SemaphoreType.DMA,
        ],
    )(x_pad, m, k_eff.astype(jnp.int32), top_ps.astype(jnp.float32))

    sel = sel.reshape(B, 4)
    tk, jk, tp, jp = sel[:, 0], sel[:, 1], sel[:, 2], sel[:, 3]

    g_pad = jnp.pad(g, ((0, 0), (0, VP - V)), constant_values=0.0)
    out = pl.pallas_call(
        _final_body,
        grid=(nblk,),
        in_specs=[row_spec, row_spec, s_spec, s_spec, s_spec, s_spec,
                  s_spec, s_spec],
        out_specs=s_spec,
        out_shape=jax.ShapeDtypeStruct((nblk, 1, rb), jnp.int32),
    )(x_pad, g_pad, m3, r3(tk, jnp.int32), r3(jk, jnp.int32),
      r3(tp, jnp.int32), r3(jp, jnp.int32), r3(min_ps, jnp.float32))
    return out.reshape(B)


def kernel(logits, temperatures, top_ps, top_ks, min_ps):
    t = jnp.maximum(temperatures, MIN_TEMPERATURE)
    g = jax.random.gumbel(jax.random.key(123), (B, V), jnp.float32)
    return _run(logits.astype(jnp.float32), t, top_ps, top_ks, min_ps, g)


# speedup vs baseline: 1.1337x; 1.0305x over previous
import numpy as np
import jax
import jax.numpy as jnp
from jax import lax
from jax.experimental import pallas as pl
from jax.experimental.pallas import tpu as pltpu
from jax.experimental.pallas import tpu_sc as plsc

MIN_TEMPERATURE = np.float32(1e-8)
LOG_MIN_PROB = np.float32(np.log(np.float32(1e-10)))
INT_MIN = np.int32(-2**31)
INT_MAX = np.int32(2**31 - 1)
ROW_BLOCK = 8
B, V = 128, 100000
CHUNK = 8192
NCHUNK = 13
VP = CHUNK * NCHUNK
NBKT = 2048
BSHIFT = 21
CAP = 8192
NW = 32
ROWS_PER_W = B // NW


def _floor_avg(lo, hi):
    return (lo >> 1) + (hi >> 1) + (lo & hi & 1)


def _fold(ibits):
    return ibits ^ ((ibits >> 31) & np.int32(0x7FFFFFFF))


def _prep_body(logits_ref, t_ref, x_ref, m_ref):
    rb = logits_ref.shape[0]
    t = t_ref[0, 0, :].reshape(rb, 1)
    x = logits_ref[...] / t
    x = x + np.float32(0.0)
    x_ref[...] = x
    m_ref[0, 0, :] = jnp.max(x, axis=-1)


def _sc_iota():
    return lax.iota(jnp.int32, 16)


def _splat_i(v):
    return jnp.full((16,), 0, jnp.int32) + v


def _sload(ref, i):
    v = ref[pl.ds(i, 16)]
    z = jnp.zeros((16,), v.dtype)
    return jnp.sum(jnp.where(_sc_iota() == 0, v, z))


def _select_body(x_hbm, m_hbm, keff_hbm, topp_hbm, out_hbm,
                 xb, xb2, hcnt, hsum, mcnt, msum, sufcnt, sufsum,
                 keybuf, idxbuf, sm, skeff, stp, resbuf, dsem, dsem2):
    wid = lax.axis_index("s") * 2 + lax.axis_index("c")
    pltpu.sync_copy(m_hbm, sm.at[pl.ds(0, B)])
    pltpu.sync_copy(keff_hbm, skeff.at[pl.ds(0, B)])
    pltpu.sync_copy(topp_hbm, stp.at[pl.ds(0, B)])
    lanes = _sc_iota()
    ones_i = jnp.full((16,), 1, jnp.int32)
    zeros_f = jnp.zeros((16,), jnp.float32)
    zeros_i = jnp.zeros((16,), jnp.int32)

    def row_pass(row, vreg_fn, carry_init):
        bufs = (xb, xb2)
        sems = (dsem, dsem2)

        def start(c):
            return pltpu.async_copy(
                x_hbm.at[row, pl.ds(c * CHUNK, CHUNK)], bufs[c % 2],
                sems[c % 2])

        copy = start(0)
        carry = carry_init
        for c in range(NCHUNK):
            nxt = start(c + 1) if c + 1 < NCHUNK else None
            copy.wait()
            buf = bufs[c % 2]

            def vbody(v, cc, c=c, buf=buf):
                for j in range(4):
                    xv = buf[pl.ds((v * 4 + j) * 16, 16)]
                    cc = vreg_fn(c * CHUNK + (v * 4 + j) * 16, xv, cc)
                return cc

            carry = lax.fori_loop(0, CHUNK // 64, vbody, carry)
            copy = nxt
        return carry

    def process_row(r, _):
        row = wid * ROWS_PER_W + r
        m_row = _sload(sm, row)
        keff = _sload(skeff, row)
        topp = _sload(stp, row)

        def clr(i, _c):
            hcnt[pl.ds(i * 16, 16)] = zeros_i
            hsum[pl.ds(i * 16, 16)] = zeros_f
            return 0

        lax.fori_loop(0, (NBKT * 16) // 16, clr, 0)

        def hist_fn(base, xv, c):
            e = jnp.exp(xv - m_row)
            sk = _fold(lax.bitcast_convert_type(xv, jnp.int32))
            ub = lax.shift_right_logical(sk ^ INT_MIN, BSHIFT)
            addr = ub * 16 + lanes
            plsc.addupdate_scatter(hcnt, [addr], ones_i)
            plsc.addupdate_scatter(hsum, [addr], e)
            return c

        row_pass(row, hist_fn, 0)

        def merge(ci, _c):
            ac = zeros_i
            asm = zeros_f
            for j in range(16):
                b = ci * 16 + j
                cs = jnp.sum(hcnt[pl.ds(b * 16, 16)])
                ss = jnp.sum(hsum[pl.ds(b * 16, 16)])
                ac = jnp.where(lanes == j, cs, ac)
                asm = jnp.where(lanes == j, ss, asm)
            mcnt[pl.ds(ci * 16, 16)] = ac
            msum[pl.ds(ci * 16, 16)] = asm
            return 0

        lax.fori_loop(0, NBKT // 16, merge, 0)

        def sfx(cj, carry):
            rc, rs = carry
            ci = NBKT // 16 - 1 - cj
            cv = lax.rev(mcnt[pl.ds(ci * 16, 16)], (0,))
            sv = lax.rev(msum[pl.ds(ci * 16, 16)], (0,))
            cc = plsc.cumsum(cv)
            cs = plsc.cumsum(sv)
            sufcnt[pl.ds(ci * 16, 16)] = lax.rev(rc + cc - cv, (0,))
            sufsum[pl.ds(ci * 16, 16)] = lax.rev(rs + cs - sv, (0,))
            return rc + jnp.max(cc), rs + jnp.max(cs)

        lax.fori_loop(0, NBKT // 16, sfx, (np.int32(0), np.float32(0.0)))

        def bk_scan(ci, bk):
            sc_v = sufcnt[pl.ds(ci * 16, 16)]
            bidx = ci * 16 + lanes
            cand = jnp.where(sc_v < keff, bidx, NBKT)
            return jnp.minimum(bk, jnp.min(cand))

        bk = lax.fori_loop(0, NBKT // 16, bk_scan, np.int32(NBKT))
        n_gt_above = _sload(sufcnt, bk)
        sum_above = _sload(sufsum, bk)

        def compact(bkt):
            def cfn(base, xv, coff):
                sk = _fold(lax.bitcast_convert_type(xv, jnp.int32))
                ub = lax.shift_right_logical(sk ^ INT_MIN, BSHIFT)
                gidx = base + lanes
                mask = (ub == bkt) & (gidx < V) & (coff < CAP)
                plsc.store_compressed(keybuf.at[pl.ds(coff, 16)], sk,
                                      mask=mask)
                plsc.store_compressed(idxbuf.at[pl.ds(coff, 16)], gidx,
                                      mask=mask)
                return coff + jnp.sum(mask.astype(jnp.int32))

            return row_pass(row, cfn, np.int32(0))

        ck = compact(bk)
        ktrips = (ck + 15) >> 4

        def masked_count_gt(mid):
            def body(i, acc):
                kv = keybuf[pl.ds(i * 16, 16)]
                valid = (i * 16 + lanes) < ck
                return acc + jnp.where((kv > mid) & valid, ones_i, zeros_i)

            return jnp.sum(lax.fori_loop(0, ktrips, body, zeros_i))

        lo0 = (bk << BSHIFT) ^ INT_MIN
        hi0 = lo0 | np.int32((1 << BSHIFT) - 1)

        def tk_it(_, lh):
            lo, hi = lh
            mid = _floor_avg(lo, hi)
            pred = (n_gt_above + masked_count_gt(mid)) >= keff
            return jnp.where(pred, mid + 1, lo), jnp.where(pred, hi, mid)

        tk, _ = lax.fori_loop(0, BSHIFT, tk_it, (lo0, hi0))
        r_k = keff - (n_gt_above + masked_count_gt(tk))

        def sgt_in(i, acc):
            kv = keybuf[pl.ds(i * 16, 16)]
            valid = (i * 16 + lanes) < ck
            ib = _fold(kv)
            ev = jnp.exp(lax.bitcast_convert_type(ib, jnp.float32) - m_row)
            return acc + jnp.where((kv > tk) & valid, ev, zeros_f)

        sum_gt_tk = jnp.sum(lax.fori_loop(0, ktrips, sgt_in, zeros_f))

        def jscan(tkey, rwant, kept_extra_tk, jk_arg):
            def body(i, carry):
                bs, jfound = carry
                kv = keybuf[pl.ds(i * 16, 16)]
                iv = idxbuf[pl.ds(i * 16, 16)]
                valid = (i * 16 + lanes) < ck
                keep1 = (kv > kept_extra_tk) | ((kv == kept_extra_tk) &
                                                (iv <= jk_arg))
                eq = (kv == tkey) & valid & keep1
                pc = plsc.cumsum(eq.astype(jnp.int32)) + bs
                hit = eq & (pc == rwant)
                jf = jnp.max(jnp.where(hit, iv, -1))
                return jnp.max(pc), jnp.maximum(jfound, jf)

            _, j = lax.fori_loop(0, ktrips, body, (np.int32(0), np.int32(-1)))
            return j

        jk = jscan(tk, r_k, tk, INT_MAX)

        e_tk = jnp.max(jnp.exp(
            lax.bitcast_convert_type(_fold(_splat_i(tk)), jnp.float32)
            - m_row))
        z1 = sum_above + sum_gt_tk + r_k.astype(jnp.float32) * e_tk
        thresh = topp * z1

        def bp_scan(ci, bp):
            sv = sufsum[pl.ds(ci * 16, 16)]
            mc = mcnt[pl.ds(ci * 16, 16)]
            bidx = ci * 16 + lanes
            pred = (sv <= thresh) & ((mc > 0) | (bidx == bk)) & (bidx >= bk)
            cand = jnp.where(pred, bidx, NBKT)
            return jnp.minimum(bp, jnp.min(cand))

        bp = lax.fori_loop(0, NBKT // 16, bp_scan, np.int32(NBKT))
        sum_above_p = _sload(sufsum, bp)

        cp = compact(bp)
        ptrips = (cp + 15) >> 4

        def masked_sum_gt_kept(mid):
            def body(i, acc):
                kv = keybuf[pl.ds(i * 16, 16)]
                iv = idxbuf[pl.ds(i * 16, 16)]
                valid = (i * 16 + lanes) < cp
                keep1 = (kv > tk) | ((kv == tk) & (iv <= jk))
                ib = _fold(kv)
                ev = jnp.exp(lax.bitcast_convert_type(ib, jnp.float32)
                             - m_row)
                return acc + jnp.where((kv > mid) & valid & keep1, ev,
                                       zeros_f)

            return jnp.sum(lax.fori_loop(0, ptrips, body, zeros_f))

        lo0p = (bp << BSHIFT) ^ INT_MIN
        hi0p = lo0p | np.int32((1 << BSHIFT) - 1)

        def tp_it(_, lh):
            lo, hi = lh
            mid = _floor_avg(lo, hi)
            pred = (sum_above_p + masked_sum_gt_kept(mid)) <= thresh
            return jnp.where(pred, lo, mid + 1), jnp.where(pred, mid, hi)

        tp, _ = lax.fori_loop(0, BSHIFT, tp_it, (lo0p, hi0p))

        d_t = sum_above_p + masked_sum_gt_kept(tp)

        def neq_kept(i, acc):
            kv = keybuf[pl.ds(i * 16, 16)]
            iv = idxbuf[pl.ds(i * 16, 16)]
            valid = (i * 16 + lanes) < cp
            keep1 = (kv > tk) | ((kv == tk) & (iv <= jk))
            return acc + jnp.where((kv == tp) & valid & keep1, ones_i,
                                   zeros_i)

        n_eq_p = jnp.sum(lax.fori_loop(0, ptrips, neq_kept, zeros_i))
        e_tp = jnp.max(jnp.exp(
            lax.bitcast_convert_type(_fold(_splat_i(tp)), jnp.float32)
            - m_row))
        q = jnp.max((zeros_f + (thresh - d_t)) / (zeros_f + e_tp))
        r_p = jnp.minimum(q, n_eq_p.astype(jnp.float32)).astype(jnp.int32) + 1
        r_p = jnp.minimum(r_p, n_eq_p)
        r_p = jnp.where(e_tp > np.float32(0.0), r_p, n_eq_p)

        def jscan_p():
            def body(i, carry):
                bs, jfound = carry
                kv = keybuf[pl.ds(i * 16, 16)]
                iv = idxbuf[pl.ds(i * 16, 16)]
                valid = (i * 16 + lanes) < cp
                keep1 = (kv > tk) | ((kv == tk) & (iv <= jk))
                eq = (kv == tp) & valid & keep1
                pc = plsc.cumsum(eq.astype(jnp.int32)) + bs
                hit = eq & (pc == r_p)
                jf = jnp.max(jnp.where(hit, iv, -1))
                return jnp.max(pc), jnp.maximum(jfound, jf)

            _, j = lax.fori_loop(0, ptrips, body,
                                 (np.int32(0), np.int32(-1)))
            return j

        jp = jscan_p()

        vals = jnp.where(lanes % 4 == 0, tk,
                         jnp.where(lanes % 4 == 1, jk,
                                   jnp.where(lanes % 4 == 2, tp, jp)))
        plsc.store_scatter(resbuf, [jnp.minimum(r * 4 + lanes, 15)], vals,
                           mask=lanes < 4)
        return 0

    lax.fori_loop(0, ROWS_PER_W, process_row, 0)
    pltpu.sync_copy(resbuf, out_hbm.at[wid])


def _final_body(x_ref, g_ref, m_ref, tk_ref, jk_ref, tp_ref, jp_ref, mp_ref,
                out_ref):
    rb, vp = x_ref.shape
    m = m_ref[0, 0, :].reshape(rb, 1)
    tk = tk_ref[0, 0, :].reshape(rb, 1)
    jk = jk_ref[0, 0, :].reshape(rb, 1)
    tp = tp_ref[0, 0, :].reshape(rb, 1)
    jp = jp_ref[0, 0, :].reshape(rb, 1)
    min_p = mp_ref[0, 0, :].reshape(rb, 1)

    x = x_ref[...]
    skey = _fold(lax.bitcast_convert_type(x, jnp.int32))
    iota = lax.broadcasted_iota(jnp.int32, (rb, vp), 1)
    e = jnp.exp(x - m)

    kept2 = ((skey > tk) | ((skey == tk) & (iota <= jk))) & \
            ((skey > tp) | ((skey == tp) & (iota <= jp)))
    z2 = jnp.sum(jnp.where(kept2, e, np.float32(0.0)), axis=-1,
                 keepdims=True)
    thr = min_p * (np.float32(1.0) / z2)
    kept3 = kept2 & jnp.logical_not((e / z2) < thr)
    z3 = jnp.sum(jnp.where(kept3, e, np.float32(0.0)), axis=-1,
                 keepdims=True)
    lz3 = jnp.log(z3)
    lp = jnp.where(kept3, jnp.maximum(x - m - lz3, LOG_MIN_PROB),
                   LOG_MIN_PROB)
    f = g_ref[...] + lp
    fmax = jnp.max(f, axis=-1, keepdims=True)
    tok = jnp.min(jnp.where(f == fmax, iota, vp), axis=-1)
    out_ref[0, 0, :] = tok


def _run(logits, t, top_ps, top_ks, min_ps, g):
    rb = ROW_BLOCK
    nblk = B // rb

    def r3(a, dtype):
        return a.astype(dtype).reshape(nblk, 1, rb)

    row_spec = pl.BlockSpec((rb, VP), lambda i: (i, 0))
    s_spec = pl.BlockSpec((1, 1, rb), lambda i: (i, 0, 0))

    logits_p = jnp.pad(logits, ((0, 0), (0, VP - V)),
                       constant_values=-np.inf)

    x_pad, m3 = pl.pallas_call(
        _prep_body,
        grid=(nblk,),
        in_specs=[row_spec, s_spec],
        out_specs=[row_spec, s_spec],
        out_shape=[jax.ShapeDtypeStruct((B, VP), jnp.float32),
                   jax.ShapeDtypeStruct((nblk, 1, rb), jnp.float32)],
    )(logits_p, r3(t, jnp.float32))
    m = m3.reshape(B)

    k_eff = jnp.where((top_ks > 0) & (top_ks < V), top_ks, V)

    mesh = plsc.VectorSubcoreMesh(core_axis_name="c", subcore_axis_name="s")
    sel = pl.kernel(
        _select_body,
        mesh=mesh,
        out_type=jax.ShapeDtypeStruct((NW, 16), jnp.int32),
        compiler_params=pltpu.CompilerParams(needs_layout_passes=False),
        scratch_types=[
            pltpu.VMEM((CHUNK,), jnp.float32),
            pltpu.VMEM((CHUNK,), jnp.float32),
            pltpu.VMEM((NBKT * 16,), jnp.int32),
            pltpu.VMEM((NBKT * 16,), jnp.float32),
            pltpu.VMEM((NBKT,), jnp.int32),
            pltpu.VMEM((NBKT,), jnp.float32),
            pltpu.VMEM((NBKT + 16,), jnp.int32),
            pltpu.VMEM((NBKT + 16,), jnp.float32),
            pltpu.VMEM((CAP + 16,), jnp.int32),
            pltpu.VMEM((CAP + 16,), jnp.int32),
            pltpu.VMEM((B + 16,), jnp.float32),
            pltpu.VMEM((B + 16,), jnp.int32),
            pltpu.VMEM((B + 16,), jnp.float32),
            pltpu.VMEM((16,), jnp.int32),
            pltpu.SemaphoreType.DMA,
            pltpu.SemaphoreType.DMA,
        ],
    )(x_pad, m, k_eff.astype(jnp.int32), top_ps.astype(jnp.float32))

    sel = sel.reshape(B, 4)
    tk, jk, tp, jp = sel[:, 0], sel[:, 1], sel[:, 2], sel[:, 3]

    g_pad = jnp.pad(g, ((0, 0), (0, VP - V)), constant_values=0.0)
    out = pl.pallas_call(
        _final_body,
        grid=(nblk,),
        in_specs=[row_spec, row_spec, s_spec, s_spec, s_spec, s_spec,
                  s_spec, s_spec],
        out_specs=s_spec,
        out_shape=jax.ShapeDtypeStruct((nblk, 1, rb), jnp.int32),
    )(x_pad, g_pad, m3, r3(tk, jnp.int32), r3(jk, jnp.int32),
      r3(tp, jnp.int32), r3(jp, jnp.int32), r3(min_ps, jnp.float32))
    return out.reshape(B)


def kernel(logits, temperatures, top_ps, top_ks, min_ps):
    t = jnp.maximum(temperatures, MIN_TEMPERATURE)
    g = jax.random.gumbel(jax.random.key(123), (B, V), jnp.float32)
    return _run(logits.astype(jnp.float32), t, top_ps, top_ks, min_ps, g)


# speedup vs baseline: 1.1541x; 1.0180x over previous
import numpy as np
import jax
import jax.numpy as jnp
from jax import lax
from jax.experimental import pallas as pl
from jax.experimental.pallas import tpu as pltpu
from jax.experimental.pallas import tpu_sc as plsc

MIN_TEMPERATURE = np.float32(1e-8)
LOG_MIN_PROB = np.float32(np.log(np.float32(1e-10)))
INT_MIN = np.int32(-2**31)
INT_MAX = np.int32(2**31 - 1)
ROW_BLOCK = 8
B, V = 128, 100000
CHUNK = 8192
NCHUNK = 13
VP = CHUNK * NCHUNK
NBKT = 2048
BSHIFT = 21
CAP = 8192
NW = 32
ROWS_PER_W = B // NW


def _floor_avg(lo, hi):
    return (lo >> 1) + (hi >> 1) + (lo & hi & 1)


def _fold(ibits):
    return ibits ^ ((ibits >> 31) & np.int32(0x7FFFFFFF))


def _prep_body(logits_ref, t_ref, x_ref, e_ref, m_ref):
    rb = logits_ref.shape[0]
    t = t_ref[0, 0, :].reshape(rb, 1)
    x = logits_ref[...] / t
    x = x + np.float32(0.0)
    x_ref[...] = x
    m = jnp.max(x, axis=-1, keepdims=True)
    e_ref[...] = jnp.exp(x - m)
    m_ref[0, 0, :] = m.reshape(rb)


def _sc_iota():
    return lax.iota(jnp.int32, 16)


def _splat_i(v):
    return jnp.full((16,), 0, jnp.int32) + v


def _sload(ref, i):
    v = ref[pl.ds(i, 16)]
    z = jnp.zeros((16,), v.dtype)
    return jnp.sum(jnp.where(_sc_iota() == 0, v, z))


def _select_body(x_hbm, e_hbm, m_hbm, keff_hbm, topp_hbm, out_hbm,
                 xb, xb2, eb, eb2, hcnt, hsum, mcnt, msum, sufcnt, sufsum,
                 keybuf, idxbuf, sm, skeff, stp, resbuf,
                 dsem, dsem2, esem, esem2):
    wid = lax.axis_index("s") * 2 + lax.axis_index("c")
    pltpu.sync_copy(m_hbm, sm.at[pl.ds(0, B)])
    pltpu.sync_copy(keff_hbm, skeff.at[pl.ds(0, B)])
    pltpu.sync_copy(topp_hbm, stp.at[pl.ds(0, B)])
    lanes = _sc_iota()
    ones_i = jnp.full((16,), 1, jnp.int32)
    zeros_f = jnp.zeros((16,), jnp.float32)
    zeros_i = jnp.zeros((16,), jnp.int32)

    def row_pass(row, vreg_fn, carry_init, with_e=False):
        bufs = (xb, xb2)
        sems = (dsem, dsem2)
        ebufs = (eb, eb2)
        esems = (esem, esem2)

        def start(c):
            sl = pl.ds(c * CHUNK, CHUNK)
            h = [pltpu.async_copy(x_hbm.at[row, sl], bufs[c % 2],
                                  sems[c % 2])]
            if with_e:
                h.append(pltpu.async_copy(e_hbm.at[row, sl], ebufs[c % 2],
                                          esems[c % 2]))
            return h

        copy = start(0)
        carry = carry_init
        for c in range(NCHUNK):
            nxt = start(c + 1) if c + 1 < NCHUNK else None
            for h in copy:
                h.wait()
            buf = bufs[c % 2]
            ebuf = ebufs[c % 2]

            def vbody(v, cc, c=c, buf=buf, ebuf=ebuf):
                for j in range(4):
                    sl = pl.ds((v * 4 + j) * 16, 16)
                    xv = buf[sl]
                    ev = ebuf[sl] if with_e else None
                    cc = vreg_fn(c * CHUNK + (v * 4 + j) * 16, xv, ev, cc)
                return cc

            carry = lax.fori_loop(0, CHUNK // 64, vbody, carry)
            copy = nxt
        return carry

    def process_row(r, _):
        row = wid * ROWS_PER_W + r
        m_row = _sload(sm, row)
        keff = _sload(skeff, row)
        topp = _sload(stp, row)

        def clr(i, _c):
            hcnt[pl.ds(i * 16, 16)] = zeros_i
            hsum[pl.ds(i * 16, 16)] = zeros_f
            return 0

        lax.fori_loop(0, (NBKT * 16) // 16, clr, 0)

        def hist_fn(base, xv, ev, c):
            sk = _fold(lax.bitcast_convert_type(xv, jnp.int32))
            ub = lax.shift_right_logical(sk ^ INT_MIN, BSHIFT)
            addr = ub * 16 + lanes
            plsc.addupdate_scatter(hcnt, [addr], ones_i)
            plsc.addupdate_scatter(hsum, [addr], ev)
            return c

        row_pass(row, hist_fn, 0, with_e=True)

        def merge(ci, _c):
            ac = zeros_i
            asm = zeros_f
            for j in range(16):
                b = ci * 16 + j
                cs = jnp.sum(hcnt[pl.ds(b * 16, 16)])
                ss = jnp.sum(hsum[pl.ds(b * 16, 16)])
                ac = jnp.where(lanes == j, cs, ac)
                asm = jnp.where(lanes == j, ss, asm)
            mcnt[pl.ds(ci * 16, 16)] = ac
            msum[pl.ds(ci * 16, 16)] = asm
            return 0

        lax.fori_loop(0, NBKT // 16, merge, 0)

        def sfx(cj, carry):
            rc, rs = carry
            ci = NBKT // 16 - 1 - cj
            cv = lax.rev(mcnt[pl.ds(ci * 16, 16)], (0,))
            sv = lax.rev(msum[pl.ds(ci * 16, 16)], (0,))
            cc = plsc.cumsum(cv)
            cs = plsc.cumsum(sv)
            sufcnt[pl.ds(ci * 16, 16)] = lax.rev(rc + cc - cv, (0,))
            sufsum[pl.ds(ci * 16, 16)] = lax.rev(rs + cs - sv, (0,))
            return rc + jnp.max(cc), rs + jnp.max(cs)

        lax.fori_loop(0, NBKT // 16, sfx, (np.int32(0), np.float32(0.0)))

        def bk_scan(ci, bk):
            sc_v = sufcnt[pl.ds(ci * 16, 16)]
            bidx = ci * 16 + lanes
            cand = jnp.where(sc_v < keff, bidx, NBKT)
            return jnp.minimum(bk, jnp.min(cand))

        bk = lax.fori_loop(0, NBKT // 16, bk_scan, np.int32(NBKT))
        n_gt_above = _sload(sufcnt, bk)
        sum_above = _sload(sufsum, bk)

        def compact(bkt):
            def cfn(base, xv, ev, coff):
                sk = _fold(lax.bitcast_convert_type(xv, jnp.int32))
                ub = lax.shift_right_logical(sk ^ INT_MIN, BSHIFT)
                gidx = base + lanes
                mask = (ub == bkt) & (gidx < V) & (coff < CAP)
                plsc.store_compressed(keybuf.at[pl.ds(coff, 16)], sk,
                                      mask=mask)
                plsc.store_compressed(idxbuf.at[pl.ds(coff, 16)], gidx,
                                      mask=mask)
                return coff + jnp.sum(mask.astype(jnp.int32))

            return row_pass(row, cfn, np.int32(0))

        ck = compact(bk)
        ktrips = (ck + 15) >> 4

        def masked_count_gt(mid):
            def body(i, acc):
                kv = keybuf[pl.ds(i * 16, 16)]
                valid = (i * 16 + lanes) < ck
                return acc + jnp.where((kv > mid) & valid, ones_i, zeros_i)

            return jnp.sum(lax.fori_loop(0, ktrips, body, zeros_i))

        lo0 = (bk << BSHIFT) ^ INT_MIN
        hi0 = lo0 | np.int32((1 << BSHIFT) - 1)

        def tk_it(_, lh):
            lo, hi = lh
            mid = _floor_avg(lo, hi)
            pred = (n_gt_above + masked_count_gt(mid)) >= keff
            return jnp.where(pred, mid + 1, lo), jnp.where(pred, hi, mid)

        tk, _ = lax.fori_loop(0, BSHIFT, tk_it, (lo0, hi0))
        r_k = keff - (n_gt_above + masked_count_gt(tk))

        def sgt_in(i, acc):
            kv = keybuf[pl.ds(i * 16, 16)]
            valid = (i * 16 + lanes) < ck
            ib = _fold(kv)
            ev = jnp.exp(lax.bitcast_convert_type(ib, jnp.float32) - m_row)
            return acc + jnp.where((kv > tk) & valid, ev, zeros_f)

        sum_gt_tk = jnp.sum(lax.fori_loop(0, ktrips, sgt_in, zeros_f))

        def jscan(tkey, rwant, kept_extra_tk, jk_arg):
            def body(i, carry):
                bs, jfound = carry
                kv = keybuf[pl.ds(i * 16, 16)]
                iv = idxbuf[pl.ds(i * 16, 16)]
                valid = (i * 16 + lanes) < ck
                keep1 = (kv > kept_extra_tk) | ((kv == kept_extra_tk) &
                                                (iv <= jk_arg))
                eq = (kv == tkey) & valid & keep1
                pc = plsc.cumsum(eq.astype(jnp.int32)) + bs
                hit = eq & (pc == rwant)
                jf = jnp.max(jnp.where(hit, iv, -1))
                return jnp.max(pc), jnp.maximum(jfound, jf)

            _, j = lax.fori_loop(0, ktrips, body, (np.int32(0), np.int32(-1)))
            return j

        jk = jscan(tk, r_k, tk, INT_MAX)

        e_tk = jnp.max(jnp.exp(
            lax.bitcast_convert_type(_fold(_splat_i(tk)), jnp.float32)
            - m_row))
        z1 = sum_above + sum_gt_tk + r_k.astype(jnp.float32) * e_tk
        thresh = topp * z1

        def bp_scan(ci, bp):
            sv = sufsum[pl.ds(ci * 16, 16)]
            mc = mcnt[pl.ds(ci * 16, 16)]
            bidx = ci * 16 + lanes
            pred = (sv <= thresh) & ((mc > 0) | (bidx == bk)) & (bidx >= bk)
            cand = jnp.where(pred, bidx, NBKT)
            return jnp.minimum(bp, jnp.min(cand))

        bp = lax.fori_loop(0, NBKT // 16, bp_scan, np.int32(NBKT))
        sum_above_p = _sload(sufsum, bp)

        cp = compact(bp)
        ptrips = (cp + 15) >> 4

        def masked_sum_gt_kept(mid):
            def body(i, acc):
                kv = keybuf[pl.ds(i * 16, 16)]
                iv = idxbuf[pl.ds(i * 16, 16)]
                valid = (i * 16 + lanes) < cp
                keep1 = (kv > tk) | ((kv == tk) & (iv <= jk))
                ib = _fold(kv)
                ev = jnp.exp(lax.bitcast_convert_type(ib, jnp.float32)
                             - m_row)
                return acc + jnp.where((kv > mid) & valid & keep1, ev,
                                       zeros_f)

            return jnp.sum(lax.fori_loop(0, ptrips, body, zeros_f))

        lo0p = (bp << BSHIFT) ^ INT_MIN
        hi0p = lo0p | np.int32((1 << BSHIFT) - 1)

        def tp_it(_, lh):
            lo, hi = lh
            mid = _floor_avg(lo, hi)
            pred = (sum_above_p + masked_sum_gt_kept(mid)) <= thresh
            return jnp.where(pred, lo, mid + 1), jnp.where(pred, mid, hi)

        tp, _ = lax.fori_loop(0, BSHIFT, tp_it, (lo0p, hi0p))

        d_t = sum_above_p + masked_sum_gt_kept(tp)

        def neq_kept(i, acc):
            kv = keybuf[pl.ds(i * 16, 16)]
            iv = idxbuf[pl.ds(i * 16, 16)]
            valid = (i * 16 + lanes) < cp
            keep1 = (kv > tk) | ((kv == tk) & (iv <= jk))
            return acc + jnp.where((kv == tp) & valid & keep1, ones_i,
                                   zeros_i)

        n_eq_p = jnp.sum(lax.fori_loop(0, ptrips, neq_kept, zeros_i))
        e_tp = jnp.max(jnp.exp(
            lax.bitcast_convert_type(_fold(_splat_i(tp)), jnp.float32)
            - m_row))
        q = jnp.max((zeros_f + (thresh - d_t)) / (zeros_f + e_tp))
        r_p = jnp.minimum(q, n_eq_p.astype(jnp.float32)).astype(jnp.int32) + 1
        r_p = jnp.minimum(r_p, n_eq_p)
        r_p = jnp.where(e_tp > np.float32(0.0), r_p, n_eq_p)

        def jscan_p():
            def body(i, carry):
                bs, jfound = carry
                kv = keybuf[pl.ds(i * 16, 16)]
                iv = idxbuf[pl.ds(i * 16, 16)]
                valid = (i * 16 + lanes) < cp
                keep1 = (kv > tk) | ((kv == tk) & (iv <= jk))
                eq = (kv == tp) & valid & keep1
                pc = plsc.cumsum(eq.astype(jnp.int32)) + bs
                hit = eq & (pc == r_p)
                jf = jnp.max(jnp.where(hit, iv, -1))
                return jnp.max(pc), jnp.maximum(jfound, jf)

            _, j = lax.fori_loop(0, ptrips, body,
                                 (np.int32(0), np.int32(-1)))
            return j

        jp = jscan_p()

        vals = jnp.where(lanes % 4 == 0, tk,
                         jnp.where(lanes % 4 == 1, jk,
                                   jnp.where(lanes % 4 == 2, tp, jp)))
        plsc.store_scatter(resbuf, [jnp.minimum(r * 4 + lanes, 15)], vals,
                           mask=lanes < 4)
        return 0

    lax.fori_loop(0, ROWS_PER_W, process_row, 0)
    pltpu.sync_copy(resbuf, out_hbm.at[wid])


def _final_body(x_ref, g_ref, m_ref, tk_ref, jk_ref, tp_ref, jp_ref, mp_ref,
                out_ref):
    rb, vp = x_ref.shape
    m = m_ref[0, 0, :].reshape(rb, 1)
    tk = tk_ref[0, 0, :].reshape(rb, 1)
    jk = jk_ref[0, 0, :].reshape(rb, 1)
    tp = tp_ref[0, 0, :].reshape(rb, 1)
    jp = jp_ref[0, 0, :].reshape(rb, 1)
    min_p = mp_ref[0, 0, :].reshape(rb, 1)

    x = x_ref[...]
    skey = _fold(lax.bitcast_convert_type(x, jnp.int32))
    iota = lax.broadcasted_iota(jnp.int32, (rb, vp), 1)
    e = jnp.exp(x - m)

    kept2 = ((skey > tk) | ((skey == tk) & (iota <= jk))) & \
            ((skey > tp) | ((skey == tp) & (iota <= jp)))
    z2 = jnp.sum(jnp.where(kept2, e, np.float32(0.0)), axis=-1,
                 keepdims=True)
    thr = min_p * (np.float32(1.0) / z2)
    kept3 = kept2 & jnp.logical_not((e / z2) < thr)
    z3 = jnp.sum(jnp.where(kept3, e, np.float32(0.0)), axis=-1,
                 keepdims=True)
    lz3 = jnp.log(z3)
    lp = jnp.where(kept3, jnp.maximum(x - m - lz3, LOG_MIN_PROB),
                   LOG_MIN_PROB)
    f = g_ref[...] + lp
    fmax = jnp.max(f, axis=-1, keepdims=True)
    tok = jnp.min(jnp.where(f == fmax, iota, vp), axis=-1)
    out_ref[0, 0, :] = tok


def _run(logits, t, top_ps, top_ks, min_ps, g):
    rb = ROW_BLOCK
    nblk = B // rb

    def r3(a, dtype):
        return a.astype(dtype).reshape(nblk, 1, rb)

    row_spec = pl.BlockSpec((rb, VP), lambda i: (i, 0))
    s_spec = pl.BlockSpec((1, 1, rb), lambda i: (i, 0, 0))

    logits_p = jnp.pad(logits, ((0, 0), (0, VP - V)),
                       constant_values=-np.inf)

    x_pad, e_pad, m3 = pl.pallas_call(
        _prep_body,
        grid=(nblk,),
        in_specs=[row_spec, s_spec],
        out_specs=[row_spec, row_spec, s_spec],
        out_shape=[jax.ShapeDtypeStruct((B, VP), jnp.float32),
                   jax.ShapeDtypeStruct((B, VP), jnp.float32),
                   jax.ShapeDtypeStruct((nblk, 1, rb), jnp.float32)],
    )(logits_p, r3(t, jnp.float32))
    m = m3.reshape(B)

    k_eff = jnp.where((top_ks > 0) & (top_ks < V), top_ks, V)

    mesh = plsc.VectorSubcoreMesh(core_axis_name="c", subcore_axis_name="s")
    sel = pl.kernel(
        _select_body,
        mesh=mesh,
        out_type=jax.ShapeDtypeStruct((NW, 16), jnp.int32),
        compiler_params=pltpu.CompilerParams(needs_layout_passes=False),
        scratch_types=[
            pltpu.VMEM((CHUNK,), jnp.float32),
            pltpu.VMEM((CHUNK,), jnp.float32),
            pltpu.VMEM((CHUNK,), jnp.float32),
            pltpu.VMEM((CHUNK,), jnp.float32),
            pltpu.VMEM((NBKT * 16,), jnp.int32),
            pltpu.VMEM((NBKT * 16,), jnp.float32),
            pltpu.VMEM((NBKT,), jnp.int32),
            pltpu.VMEM((NBKT,), jnp.float32),
            pltpu.VMEM((NBKT + 16,), jnp.int32),
            pltpu.VMEM((NBKT + 16,), jnp.float32),
            pltpu.VMEM((CAP + 16,), jnp.int32),
            pltpu.VMEM((CAP + 16,), jnp.int32),
            pltpu.VMEM((B + 16,), jnp.float32),
            pltpu.VMEM((B + 16,), jnp.int32),
            pltpu.VMEM((B + 16,), jnp.float32),
            pltpu.VMEM((16,), jnp.int32),
            pltpu.SemaphoreType.DMA,
            pltpu.SemaphoreType.DMA,
            pltpu.SemaphoreType.DMA,
            pltpu.SemaphoreType.DMA,
        ],
    )(x_pad, e_pad, m, k_eff.astype(jnp.int32),
      top_ps.astype(jnp.float32))

    sel = sel.reshape(B, 4)
    tk, jk, tp, jp = sel[:, 0], sel[:, 1], sel[:, 2], sel[:, 3]

    g_pad = jnp.pad(g, ((0, 0), (0, VP - V)), constant_values=0.0)
    out = pl.pallas_call(
        _final_body,
        grid=(nblk,),
        in_specs=[row_spec, row_spec, s_spec, s_spec, s_spec, s_spec,
                  s_spec, s_spec],
        out_specs=s_spec,
        out_shape=jax.ShapeDtypeStruct((nblk, 1, rb), jnp.int32),
    )(x_pad, g_pad, m3, r3(tk, jnp.int32), r3(jk, jnp.int32),
      r3(tp, jnp.int32), r3(jp, jnp.int32), r3(min_ps, jnp.float32))
    return out.reshape(B)


def kernel(logits, temperatures, top_ps, top_ks, min_ps):
    t = jnp.maximum(temperatures, MIN_TEMPERATURE)
    g = jax.random.gumbel(jax.random.key(123), (B, V), jnp.float32)
    return _run(logits.astype(jnp.float32), t, top_ps, top_ks, min_ps, g)
